# Initial kernel scaffold; baseline (speedup 1.0000x reference)
import jax
import jax.numpy as jnp
from jax.experimental import pallas as pl


def kernel(x, edge_index, W1, b1, W2, b2, W3, b3, W4, b4):
    raise NotImplementedError("write your pallas kernel here")


# speedup vs baseline: 7.2127x; 7.2127x over previous
import functools

import jax
import jax.numpy as jnp
from jax import lax
from jax.experimental import pallas as pl
from jax.experimental.pallas import tpu as pltpu
from jax.experimental.pallas import tpu_sc as plsc

N = 100000
E = 3200000
NC = 2
NS = 16
CH = 80
N_ACC = 102400
RPS = N_ACC // NS
ZR = 640

F32 = jnp.float32


def _sc_mesh():
    return plsc.VectorSubcoreMesh(core_axis_name="c", subcore_axis_name="s")


def _zero_acc(zb, acc, s):
    def zrow(i, _):
        zb[i, :] = jnp.zeros((16,), F32)
        return 0

    lax.fori_loop(0, ZR, zrow, 0)

    def zcopy(j, _):
        pltpu.sync_copy(zb, acc.at[pl.ds(s * RPS + j * ZR, ZR)])
        return 0

    lax.fori_loop(0, RPS // ZR, zcopy, 0)


def _spmm_body(wide, src_hbm, dst_hbm, table_hbm, out_hbm, src_v, dst_v, rows_v, zb, acc):
    c = lax.axis_index("c")
    s = lax.axis_index("s")
    _zero_acc(zb, acc, s)
    plsc.subcore_barrier()

    if wide:
        ew = E // NS
        base = s * ew
    else:
        ew = E // (NC * NS)
        base = (c * NS + s) * ew

    def body(i, _):
        off = base + i * CH
        pltpu.sync_copy(src_hbm.at[pl.ds(off, CH)], src_v)
        pltpu.sync_copy(dst_hbm.at[pl.ds(off, CH)], dst_v.at[0])
        if wide:
            shift = c * N
            for j in range(CH // 16):
                sl = pl.ds(j * 16, 16)
                src_v[sl] = src_v[sl] + shift
        pltpu.sync_copy(table_hbm.at[src_v], rows_v)
        pltpu.sync_copy(rows_v, acc.at[dst_v.at[0]], add=True)
        return 0

    lax.fori_loop(0, ew // CH, body, 0)
    plsc.subcore_barrier()
    pltpu.sync_copy(acc.at[pl.ds(s * RPS, RPS)], out_hbm.at[c, pl.ds(s * RPS, RPS)])


def _make_spmm(wide):
    table_rows = NC * N if wide else N
    return pl.kernel(
        functools.partial(_spmm_body, wide),
        out_type=jax.ShapeDtypeStruct((NC, N_ACC, 16), F32),
        mesh=_sc_mesh(),
        scratch_types=[
            pltpu.VMEM((CH,), jnp.int32),
            pltpu.VMEM((1, CH), jnp.int32),
            pltpu.VMEM((CH, 16), F32),
            pltpu.VMEM((ZR, 16), F32),
            pltpu.VMEM_SHARED((N_ACC, 16), F32),
        ],
        compiler_params=pltpu.CompilerParams(use_tc_tiling_on_sc=False),
    ), table_rows


_sc_spmm_wide, _ = _make_spmm(True)
_sc_spmm_narrow, _ = _make_spmm(False)


R = 2000
GRID = N // R


def _row_spec(width):
    return pl.BlockSpec((R, width), lambda i: (i, 0))


def _pair_spec():
    return pl.BlockSpec((NC, R, 16), lambda i: (0, i, 0))


def _full_spec(shape):
    return pl.BlockSpec(shape, lambda i: tuple(0 for _ in shape))


def _tc_call(body, in_specs, out_specs, out_shapes):
    return pl.pallas_call(
        body,
        grid=(GRID,),
        in_specs=in_specs,
        out_specs=out_specs,
        out_shape=out_shapes,
    )


def _tck0_body(deg_ref, x_ref, w1_ref, s_ref, g1_ref):
    d = deg_ref[0, :, 0:1] + deg_ref[1, :, 0:1] + 1.0
    sb = lax.rsqrt(d)
    h = jnp.dot(x_ref[...], w1_ref[...], preferred_element_type=F32)
    g = sb * h
    s_ref[...] = sb
    g1_ref[0] = g[:, :16]
    g1_ref[1] = g[:, 16:]


def _tck1_body(y_ref, g_ref, s_ref, b_ref, w_ref, xl1_ref, g2_ref):
    sb = s_ref[...]
    ycat = jnp.concatenate(
        [y_ref[0] + g_ref[0], y_ref[1] + g_ref[1]], axis=1)
    z = sb * ycat + b_ref[...][None, :]
    xl1 = jnp.maximum(z, 0.0)
    xl1_ref[...] = xl1
    g2_ref[...] = sb * jnp.dot(xl1, w_ref[...], preferred_element_type=F32)


def _tck2_body(y_ref, g_ref, s_ref, b_ref, w_ref, g3_ref):
    sb = s_ref[...]
    ysum = y_ref[0] + y_ref[1] + g_ref[...]
    z = sb * ysum + b_ref[...][None, :]
    xl2 = jnp.maximum(z, 0.0)
    g3 = sb * jnp.dot(xl2, w_ref[...], preferred_element_type=F32)
    g3_ref[0] = g3[:, :16]
    g3_ref[1] = g3[:, 16:]


def _tck3_body(y_ref, g_ref, s_ref, b_ref, xl1_ref, w_ref, g4_ref):
    sb = s_ref[...]
    ycat = jnp.concatenate(
        [y_ref[0] + g_ref[0], y_ref[1] + g_ref[1]], axis=1)
    z = sb * ycat + b_ref[...][None, :]
    xtd = jnp.maximum(z, 0.0) + xl1_ref[...]
    g4_ref[...] = sb * jnp.dot(xtd, w_ref[...], preferred_element_type=F32)


def _tck4_body(y_ref, g_ref, s_ref, b_ref, x_ref, out_ref):
    sb = s_ref[...]
    ysum = y_ref[0] + y_ref[1] + g_ref[...]
    z = sb * ysum + b_ref[...][None, :]
    out_ref[...] = jnp.maximum(z, 0.0) + x_ref[...]


def kernel(x, edge_index, W1, b1, W2, b2, W3, b3, W4, b4):
    ei = edge_index.astype(jnp.int32)
    src = ei[0]
    dst = ei[1]

    ones_tab = jnp.ones((N, 16), F32)
    deg = _sc_spmm_narrow(src, dst, ones_tab)

    s, g1s = _tc_call(
        _tck0_body,
        [_pair_spec(), _row_spec(16), _full_spec((16, 32))],
        [_row_spec(1), _pair_spec()],
        [jax.ShapeDtypeStruct((N, 1), F32),
         jax.ShapeDtypeStruct((NC, N, 16), F32)],
    )(deg, x, W1)

    y1 = _sc_spmm_wide(src, dst, g1s.reshape(NC * N, 16))

    xl1, g2 = _tc_call(
        _tck1_body,
        [_pair_spec(), _pair_spec(), _row_spec(1), _full_spec((32,)),
         _full_spec((32, 16))],
        [_row_spec(32), _row_spec(16)],
        [jax.ShapeDtypeStruct((N, 32), F32),
         jax.ShapeDtypeStruct((N, 16), F32)],
    )(y1, g1s, s, b1, W2)

    y2 = _sc_spmm_narrow(src, dst, g2)

    (g3s,) = _tc_call(
        _tck2_body,
        [_pair_spec(), _row_spec(16), _row_spec(1), _full_spec((16,)),
         _full_spec((16, 32))],
        [_pair_spec()],
        [jax.ShapeDtypeStruct((NC, N, 16), F32)],
    )(y2, g2, s, b2, W3)

    y3 = _sc_spmm_wide(src, dst, g3s.reshape(NC * N, 16))

    (g4,) = _tc_call(
        _tck3_body,
        [_pair_spec(), _pair_spec(), _row_spec(1), _full_spec((32,)),
         _row_spec(32), _full_spec((32, 16))],
        [_row_spec(16)],
        [jax.ShapeDtypeStruct((N, 16), F32)],
    )(y3, g3s, s, b3, xl1, W4)

    y4 = _sc_spmm_narrow(src, dst, g4)

    (out,) = _tc_call(
        _tck4_body,
        [_pair_spec(), _row_spec(16), _row_spec(1), _full_spec((16,)),
         _row_spec(16)],
        [_row_spec(16)],
        [jax.ShapeDtypeStruct((N, 16), F32)],
    )(y4, g4, s, b4, x)

    return out


# speedup vs baseline: 17.1505x; 2.3778x over previous
import functools

import jax
import jax.numpy as jnp
from jax import lax
from jax.experimental import pallas as pl
from jax.experimental.pallas import tpu as pltpu
from jax.experimental.pallas import tpu_sc as plsc

N = 100000
E = 3200000
NC = 2
NS = 16
CH = 80
N_ACC = N
RPS = N_ACC // NS
ZR = 625

F32 = jnp.float32


def _sc_mesh():
    return plsc.VectorSubcoreMesh(core_axis_name="c", subcore_axis_name="s")


BLK = 25
BE = BLK * CH
NSLOT = 5


def _zero_acc(zb, acc, s):
    def zrow(i, _):
        zb[i, :] = jnp.zeros((16,), F32)
        return 0

    lax.fori_loop(0, ZR, zrow, 0)

    def zcopy(j, _):
        pltpu.sync_copy(zb, acc.at[pl.ds(s * RPS + j * ZR, ZR)])
        return 0

    lax.fori_loop(0, RPS // ZR, zcopy, 0)


def _spmm_body(mode, *refs):
    if mode == "deg":
        (dst2_hbm, out_hbm, dstb, rows_v, zb, acc, si0, *sems) = refs
        src_hbm = table_hbm = srcb = None
    else:
        (src_hbm, dst2_hbm, table_hbm, out_hbm,
         srcb, dstb, rows_v, zb, acc, si0, *sems) = refs
    sg = sems[0:NSLOT]
    ss = sems[NSLOT:2 * NSLOT]

    c = lax.axis_index("c")
    s = lax.axis_index("s")
    _zero_acc(zb, acc, s)
    if mode == "deg":
        def orow(i, _):
            for p in range(NSLOT):
                rows_v[p, i, :] = jnp.full((16,), 1.0, F32)
            return 0
        lax.fori_loop(0, CH, orow, 0)
    plsc.subcore_barrier()

    if mode == "wide":
        ew = E // NS
        wbase = s * ew
    else:
        ew = E // (NC * NS)
        wbase = (c * NS + s) * ew
    nb = ew // BE
    wrow = wbase // CH

    def fire_idx(t, pb):
        if mode != "deg":
            pltpu.async_copy(src_hbm.at[pl.ds(wbase + t * BE, BE)],
                             srcb.at[pb], si0)
        pltpu.async_copy(dst2_hbm.at[pl.ds(wrow + t * BLK, BLK)],
                         dstb.at[pb], si0)

    def wait_idx(t, pb):
        if mode != "deg":
            pltpu.make_async_copy(src_hbm.at[pl.ds(wbase + t * BE, BE)],
                                  srcb.at[pb], si0).wait()
        pltpu.make_async_copy(dst2_hbm.at[pl.ds(wrow + t * BLK, BLK)],
                              dstb.at[pb], si0).wait()

    def fire_scatter(jj, p):
        t = jj // BLK
        pltpu.async_copy(rows_v.at[p], acc.at[dstb.at[t % 2, jj % BLK]],
                         ss[p], add=True)

    def wait_scatter(p):
        pltpu.make_async_copy(rows_v.at[p], acc.at[dstb.at[0, 0]],
                              ss[p]).wait()

    fire_idx(0, 0)

    def body(g5, _):
        for p in range(NSLOT):
            jj = g5 * NSLOT + p
            t = jj // BLK
            pb = t % 2
            if p == 0:
                @pl.when(g5 >= 1)
                def _():
                    jp = jj - 1
                    pltpu.make_async_copy(
                        table_hbm.at[srcb.at[0, pl.ds(0, CH)]],
                        rows_v.at[NSLOT - 1], sg[NSLOT - 1]).wait()
                    fire_scatter(jp, NSLOT - 1)

                @pl.when(g5 % 5 == 0)
                def _():
                    wait_idx(t, pb)
                    if mode == "wide":
                        shift = c * N

                        def srow(k, _):
                            sl = pl.ds(k * 16, 16)
                            srcb[pb, sl] = srcb[pb, sl] + shift
                            return 0
                        lax.fori_loop(0, BE // 16, srow, 0)

                @pl.when((g5 % 5 == 1) & (t + 1 < nb))
                def _():
                    fire_idx(t + 1, (t + 1) % 2)
            else:
                jp = jj - 1
                pltpu.make_async_copy(
                    table_hbm.at[srcb.at[0, pl.ds(0, CH)]],
                    rows_v.at[p - 1], sg[p - 1]).wait()
                fire_scatter(jp, p - 1)

            @pl.when(g5 >= 1)
            def _():
                wait_scatter(p)

            pltpu.async_copy(
                table_hbm.at[srcb.at[pb, pl.ds((jj % BLK) * CH, CH)]],
                rows_v.at[p], sg[p])
        return 0

    def body_deg(g5, _):
        for p in range(NSLOT):
            jj = g5 * NSLOT + p
            t = jj // BLK
            if p == 0:
                @pl.when(g5 % 5 == 0)
                def _():
                    wait_idx(t, t % 2)

                @pl.when((g5 % 5 == 1) & (t + 1 < nb))
                def _():
                    fire_idx(t + 1, (t + 1) % 2)

            @pl.when(g5 >= 1)
            def _():
                wait_scatter(p)

            fire_scatter(jj, p)
        return 0

    if mode == "deg":
        lax.fori_loop(0, ew // CH // NSLOT, body_deg, 0)
    else:
        lax.fori_loop(0, ew // CH // NSLOT, body, 0)
        last = ew // CH - 1
        pltpu.make_async_copy(table_hbm.at[srcb.at[0, pl.ds(0, CH)]],
                              rows_v.at[NSLOT - 1], sg[NSLOT - 1]).wait()
        fire_scatter(last, NSLOT - 1)
    for p in range(NSLOT):
        wait_scatter(p)

    plsc.subcore_barrier()
    pltpu.sync_copy(acc.at[pl.ds(s * RPS, RPS)],
                    out_hbm.at[c, pl.ds(s * RPS, RPS)])


def _make_spmm(mode):
    scratch = []
    if mode != "deg":
        scratch.append(pltpu.VMEM((2, BE), jnp.int32))
    scratch += [
        pltpu.VMEM((2, BLK, CH), jnp.int32),
        pltpu.VMEM((NSLOT, CH, 16), F32),
        pltpu.VMEM((ZR, 16), F32),
        pltpu.VMEM_SHARED((N_ACC, 16), F32),
    ]
    scratch += [pltpu.SemaphoreType.DMA] * (1 + 2 * NSLOT)
    return pl.kernel(
        functools.partial(_spmm_body, mode),
        out_type=jax.ShapeDtypeStruct((NC, N_ACC, 16), F32),
        mesh=_sc_mesh(),
        scratch_types=scratch,
        compiler_params=pltpu.CompilerParams(use_tc_tiling_on_sc=False),
    )


_sc_spmm_wide = _make_spmm("wide")
_sc_spmm_narrow = _make_spmm("narrow")
_sc_deg = _make_spmm("deg")


R = 2000
GRID = N // R


def _row_spec(width):
    return pl.BlockSpec((R, width), lambda i: (i, 0))


def _pair_spec():
    return pl.BlockSpec((NC, R, 16), lambda i: (0, i, 0))


def _full_spec(shape):
    return pl.BlockSpec(shape, lambda i: tuple(0 for _ in shape))


def _tc_call(body, in_specs, out_specs, out_shapes):
    return pl.pallas_call(
        body,
        grid=(GRID,),
        in_specs=in_specs,
        out_specs=out_specs,
        out_shape=out_shapes,
    )


def _tck0_body(deg_ref, x_ref, w1_ref, s_ref, g1_ref):
    d = deg_ref[0, :, 0:1] + deg_ref[1, :, 0:1] + 1.0
    sb = lax.rsqrt(d)
    h = jnp.dot(x_ref[...], w1_ref[...], preferred_element_type=F32)
    g = sb * h
    s_ref[...] = sb
    g1_ref[0] = g[:, :16]
    g1_ref[1] = g[:, 16:]


def _tck1_body(y_ref, g_ref, s_ref, b_ref, w_ref, xl1_ref, g2_ref):
    sb = s_ref[...]
    ycat = jnp.concatenate(
        [y_ref[0] + g_ref[0], y_ref[1] + g_ref[1]], axis=1)
    z = sb * ycat + b_ref[...][None, :]
    xl1 = jnp.maximum(z, 0.0)
    xl1_ref[...] = xl1
    g2_ref[...] = sb * jnp.dot(xl1, w_ref[...], preferred_element_type=F32)


def _tck2_body(y_ref, g_ref, s_ref, b_ref, w_ref, g3_ref):
    sb = s_ref[...]
    ysum = y_ref[0] + y_ref[1] + g_ref[...]
    z = sb * ysum + b_ref[...][None, :]
    xl2 = jnp.maximum(z, 0.0)
    g3 = sb * jnp.dot(xl2, w_ref[...], preferred_element_type=F32)
    g3_ref[0] = g3[:, :16]
    g3_ref[1] = g3[:, 16:]


def _tck3_body(y_ref, g_ref, s_ref, b_ref, xl1_ref, w_ref, g4_ref):
    sb = s_ref[...]
    ycat = jnp.concatenate(
        [y_ref[0] + g_ref[0], y_ref[1] + g_ref[1]], axis=1)
    z = sb * ycat + b_ref[...][None, :]
    xtd = jnp.maximum(z, 0.0) + xl1_ref[...]
    g4_ref[...] = sb * jnp.dot(xtd, w_ref[...], preferred_element_type=F32)


def _tck4_body(y_ref, g_ref, s_ref, b_ref, x_ref, out_ref):
    sb = s_ref[...]
    ysum = y_ref[0] + y_ref[1] + g_ref[...]
    z = sb * ysum + b_ref[...][None, :]
    out_ref[...] = jnp.maximum(z, 0.0) + x_ref[...]


def kernel(x, edge_index, W1, b1, W2, b2, W3, b3, W4, b4):
    ei = edge_index.astype(jnp.int32)
    src = ei[0]
    dst = ei[1]

    dst2 = dst.reshape(E // CH, CH)

    deg = _sc_deg(dst2)

    s, g1s = _tc_call(
        _tck0_body,
        [_pair_spec(), _row_spec(16), _full_spec((16, 32))],
        [_row_spec(1), _pair_spec()],
        [jax.ShapeDtypeStruct((N, 1), F32),
         jax.ShapeDtypeStruct((NC, N, 16), F32)],
    )(deg, x, W1)

    y1 = _sc_spmm_wide(src, dst2, g1s.reshape(NC * N, 16))

    xl1, g2 = _tc_call(
        _tck1_body,
        [_pair_spec(), _pair_spec(), _row_spec(1), _full_spec((32,)),
         _full_spec((32, 16))],
        [_row_spec(32), _row_spec(16)],
        [jax.ShapeDtypeStruct((N, 32), F32),
         jax.ShapeDtypeStruct((N, 16), F32)],
    )(y1, g1s, s, b1, W2)

    y2 = _sc_spmm_narrow(src, dst2, g2)

    (g3s,) = _tc_call(
        _tck2_body,
        [_pair_spec(), _row_spec(16), _row_spec(1), _full_spec((16,)),
         _full_spec((16, 32))],
        [_pair_spec()],
        [jax.ShapeDtypeStruct((NC, N, 16), F32)],
    )(y2, g2, s, b2, W3)

    y3 = _sc_spmm_wide(src, dst2, g3s.reshape(NC * N, 16))

    (g4,) = _tc_call(
        _tck3_body,
        [_pair_spec(), _pair_spec(), _row_spec(1), _full_spec((32,)),
         _row_spec(32), _full_spec((32, 16))],
        [_row_spec(16)],
        [jax.ShapeDtypeStruct((N, 16), F32)],
    )(y3, g3s, s, b3, xl1, W4)

    y4 = _sc_spmm_narrow(src, dst2, g4)

    (out,) = _tc_call(
        _tck4_body,
        [_pair_spec(), _row_spec(16), _row_spec(1), _full_spec((16,)),
         _row_spec(16)],
        [_row_spec(16)],
        [jax.ShapeDtypeStruct((N, 16), F32)],
    )(y4, g4, s, b4, x)

    return out


# speedup vs baseline: 42.2622x; 2.4642x over previous
import functools

import jax
import jax.numpy as jnp
from jax import lax
from jax.experimental import pallas as pl
from jax.experimental.pallas import tpu as pltpu
from jax.experimental.pallas import tpu_sc as plsc

N = 100000
E = 3200000
NC = 2
NS = 16
CH = 80
N_ACC = N
RPS = N_ACC // NS
ZR = 625

F32 = jnp.float32


def _sc_mesh():
    return plsc.VectorSubcoreMesh(core_axis_name="c", subcore_axis_name="s")


BLK = 25
BE = BLK * CH
NSLOT = 5


def _zero_acc(zb, acc, s):
    def zrow(i, _):
        zb[i, :] = jnp.zeros((16,), F32)
        return 0

    lax.fori_loop(0, ZR, zrow, 0)

    def zcopy(j, _):
        pltpu.sync_copy(zb, acc.at[pl.ds(s * RPS + j * ZR, ZR)])
        return 0

    lax.fori_loop(0, RPS // ZR, zcopy, 0)


def _spmm_body(mode, *refs):
    if mode == "deg":
        (dst2_hbm, out_hbm, dstb, rows_v, zb, acc, si0, *sems) = refs
        src_hbm = table_hbm = srcb = None
    else:
        (src_hbm, dst2_hbm, table_hbm, out_hbm,
         srcb, dstb, rows_v, zb, acc, si0, *sems) = refs
    sg = sems[0:NSLOT]
    ss = sems[NSLOT:2 * NSLOT]

    c = lax.axis_index("c")
    s = lax.axis_index("s")
    _zero_acc(zb, acc, s)
    if mode == "deg":
        def orow(i, _):
            for p in range(NSLOT):
                rows_v[p, i, :] = jnp.full((16,), 1.0, F32)
            return 0
        lax.fori_loop(0, CH, orow, 0)
    plsc.subcore_barrier()

    if mode == "wide":
        ew = E // NS
        wbase = s * ew
    else:
        ew = E // (NC * NS)
        wbase = (c * NS + s) * ew
    nb = ew // BE
    wrow = wbase // CH

    def fire_idx(t, pb):
        if mode != "deg":
            pltpu.async_copy(src_hbm.at[pl.ds(wbase + t * BE, BE)],
                             srcb.at[pb], si0)
        pltpu.async_copy(dst2_hbm.at[pl.ds(wrow + t * BLK, BLK)],
                         dstb.at[pb], si0)

    def wait_idx(t, pb):
        if mode != "deg":
            pltpu.make_async_copy(src_hbm.at[pl.ds(wbase + t * BE, BE)],
                                  srcb.at[pb], si0).wait()
        pltpu.make_async_copy(dst2_hbm.at[pl.ds(wrow + t * BLK, BLK)],
                              dstb.at[pb], si0).wait()

    def fire_scatter(jj, p):
        t = jj // BLK
        pltpu.async_copy(rows_v.at[p], acc.at[dstb.at[t % 2, jj % BLK]],
                         ss[p], add=True)

    def wait_scatter(p):
        pltpu.make_async_copy(rows_v.at[p], acc.at[dstb.at[0, 0]],
                              ss[p]).wait()

    fire_idx(0, 0)

    LAG = 3

    def body(g5, _):
        for p in range(NSLOT):
            jj = g5 * NSLOT + p
            t = jj // BLK
            pb = t % 2
            if p == 0:
                @pl.when(g5 % 5 == 0)
                def _():
                    wait_idx(t, pb)

                @pl.when((g5 % 5 == 1) & (t + 1 < nb))
                def _():
                    fire_idx(t + 1, (t + 1) % 2)

            @pl.when(g5 >= 1)
            def _():
                wait_scatter(p)

            if mode == "wide":
                shift = c * N
                cbase = (jj % BLK) * CH
                for k in range(CH // 16):
                    sl = pl.ds(cbase + k * 16, 16)
                    srcb[pb, sl] = srcb[pb, sl] + shift

            pltpu.async_copy(
                table_hbm.at[srcb.at[pb, pl.ds((jj % BLK) * CH, CH)]],
                rows_v.at[p], sg[p])

            sp = (p - LAG) % NSLOT
            cond = True if p >= LAG else (g5 >= 1)

            @pl.when(cond)
            def _():
                pltpu.make_async_copy(
                    table_hbm.at[srcb.at[0, pl.ds(0, CH)]],
                    rows_v.at[sp], sg[sp]).wait()
                fire_scatter(jj - LAG, sp)
        return 0

    def body_deg(g5, _):
        for p in range(NSLOT):
            jj = g5 * NSLOT + p
            t = jj // BLK
            if p == 0:
                @pl.when(g5 % 5 == 0)
                def _():
                    wait_idx(t, t % 2)

                @pl.when((g5 % 5 == 1) & (t + 1 < nb))
                def _():
                    fire_idx(t + 1, (t + 1) % 2)

            @pl.when(g5 >= 1)
            def _():
                wait_scatter(p)

            fire_scatter(jj, p)
        return 0

    if mode == "deg":
        lax.fori_loop(0, ew // CH // NSLOT, body_deg, 0)
    else:
        lax.fori_loop(0, ew // CH // NSLOT, body, 0)
        nch = ew // CH
        for k in range(nch - 3, nch):
            pltpu.make_async_copy(table_hbm.at[srcb.at[0, pl.ds(0, CH)]],
                                  rows_v.at[k % NSLOT], sg[k % NSLOT]).wait()
            fire_scatter(k, k % NSLOT)
    for p in range(NSLOT):
        wait_scatter(p)

    plsc.subcore_barrier()
    pltpu.sync_copy(acc.at[pl.ds(s * RPS, RPS)],
                    out_hbm.at[c, pl.ds(s * RPS, RPS)])


def _make_spmm(mode):
    scratch = []
    if mode != "deg":
        scratch.append(pltpu.VMEM((2, BE), jnp.int32))
    scratch += [
        pltpu.VMEM((2, BLK, CH), jnp.int32),
        pltpu.VMEM((NSLOT, CH, 16), F32),
        pltpu.VMEM((ZR, 16), F32),
        pltpu.VMEM_SHARED((N_ACC, 16), F32),
    ]
    scratch += [pltpu.SemaphoreType.DMA] * (1 + 2 * NSLOT)
    return pl.kernel(
        functools.partial(_spmm_body, mode),
        out_type=jax.ShapeDtypeStruct((NC, N_ACC, 16), F32),
        mesh=_sc_mesh(),
        scratch_types=scratch,
        compiler_params=pltpu.CompilerParams(use_tc_tiling_on_sc=False),
    )


_sc_spmm_wide = _make_spmm("wide")
_sc_spmm_narrow = _make_spmm("narrow")
_sc_deg = _make_spmm("deg")


R = 2000
GRID = N // R


def _row_spec(width):
    return pl.BlockSpec((R, width), lambda i: (i, 0))


def _pair_spec():
    return pl.BlockSpec((NC, R, 16), lambda i: (0, i, 0))


def _full_spec(shape):
    return pl.BlockSpec(shape, lambda i: tuple(0 for _ in shape))


def _tc_call(body, in_specs, out_specs, out_shapes):
    return pl.pallas_call(
        body,
        grid=(GRID,),
        in_specs=in_specs,
        out_specs=out_specs,
        out_shape=out_shapes,
    )


def _tck0_body(deg_ref, x_ref, w1_ref, s_ref, g1_ref):
    d = deg_ref[0, :, 0:1] + deg_ref[1, :, 0:1] + 1.0
    sb = lax.rsqrt(d)
    h = jnp.dot(x_ref[...], w1_ref[...], preferred_element_type=F32)
    g = sb * h
    s_ref[...] = sb
    g1_ref[0] = g[:, :16]
    g1_ref[1] = g[:, 16:]


def _tck1_body(y_ref, g_ref, s_ref, b_ref, w_ref, xl1_ref, g2_ref):
    sb = s_ref[...]
    ycat = jnp.concatenate(
        [y_ref[0] + g_ref[0], y_ref[1] + g_ref[1]], axis=1)
    z = sb * ycat + b_ref[...][None, :]
    xl1 = jnp.maximum(z, 0.0)
    xl1_ref[...] = xl1
    g2_ref[...] = sb * jnp.dot(xl1, w_ref[...], preferred_element_type=F32)


def _tck2_body(y_ref, g_ref, s_ref, b_ref, w_ref, g3_ref):
    sb = s_ref[...]
    ysum = y_ref[0] + y_ref[1] + g_ref[...]
    z = sb * ysum + b_ref[...][None, :]
    xl2 = jnp.maximum(z, 0.0)
    g3 = sb * jnp.dot(xl2, w_ref[...], preferred_element_type=F32)
    g3_ref[0] = g3[:, :16]
    g3_ref[1] = g3[:, 16:]


def _tck3_body(y_ref, g_ref, s_ref, b_ref, xl1_ref, w_ref, g4_ref):
    sb = s_ref[...]
    ycat = jnp.concatenate(
        [y_ref[0] + g_ref[0], y_ref[1] + g_ref[1]], axis=1)
    z = sb * ycat + b_ref[...][None, :]
    xtd = jnp.maximum(z, 0.0) + xl1_ref[...]
    g4_ref[...] = sb * jnp.dot(xtd, w_ref[...], preferred_element_type=F32)


def _tck4_body(y_ref, g_ref, s_ref, b_ref, x_ref, out_ref):
    sb = s_ref[...]
    ysum = y_ref[0] + y_ref[1] + g_ref[...]
    z = sb * ysum + b_ref[...][None, :]
    out_ref[...] = jnp.maximum(z, 0.0) + x_ref[...]


def kernel(x, edge_index, W1, b1, W2, b2, W3, b3, W4, b4):
    ei = edge_index.astype(jnp.int32)
    src = ei[0]
    dst = ei[1]

    dst2 = dst.reshape(E // CH, CH)

    deg = _sc_deg(dst2)

    s, g1s = _tc_call(
        _tck0_body,
        [_pair_spec(), _row_spec(16), _full_spec((16, 32))],
        [_row_spec(1), _pair_spec()],
        [jax.ShapeDtypeStruct((N, 1), F32),
         jax.ShapeDtypeStruct((NC, N, 16), F32)],
    )(deg, x, W1)

    y1 = _sc_spmm_wide(src, dst2, g1s.reshape(NC * N, 16))

    xl1, g2 = _tc_call(
        _tck1_body,
        [_pair_spec(), _pair_spec(), _row_spec(1), _full_spec((32,)),
         _full_spec((32, 16))],
        [_row_spec(32), _row_spec(16)],
        [jax.ShapeDtypeStruct((N, 32), F32),
         jax.ShapeDtypeStruct((N, 16), F32)],
    )(y1, g1s, s, b1, W2)

    y2 = _sc_spmm_narrow(src, dst2, g2)

    (g3s,) = _tc_call(
        _tck2_body,
        [_pair_spec(), _row_spec(16), _row_spec(1), _full_spec((16,)),
         _full_spec((16, 32))],
        [_pair_spec()],
        [jax.ShapeDtypeStruct((NC, N, 16), F32)],
    )(y2, g2, s, b2, W3)

    y3 = _sc_spmm_wide(src, dst2, g3s.reshape(NC * N, 16))

    (g4,) = _tc_call(
        _tck3_body,
        [_pair_spec(), _pair_spec(), _row_spec(1), _full_spec((32,)),
         _row_spec(32), _full_spec((32, 16))],
        [_row_spec(16)],
        [jax.ShapeDtypeStruct((N, 16), F32)],
    )(y3, g3s, s, b3, xl1, W4)

    y4 = _sc_spmm_narrow(src, dst2, g4)

    (out,) = _tc_call(
        _tck4_body,
        [_pair_spec(), _row_spec(16), _row_spec(1), _full_spec((16,)),
         _row_spec(16)],
        [_row_spec(16)],
        [jax.ShapeDtypeStruct((N, 16), F32)],
    )(y4, g4, s, b4, x)

    return out


# speedup vs baseline: 64.1561x; 1.5181x over previous
import functools

import jax
import jax.numpy as jnp
from jax import lax
from jax.experimental import pallas as pl
from jax.experimental.pallas import tpu as pltpu
from jax.experimental.pallas import tpu_sc as plsc

N = 100000
E = 3200000
NC = 2
NS = 16
CH = 80
N_ACC = 102400
RPS = N_ACC // NS
ZR = 640

F32 = jnp.float32


def _sc_mesh():
    return plsc.VectorSubcoreMesh(core_axis_name="c", subcore_axis_name="s",
                                  num_cores=NC, num_subcores=NS)


BLK = 25
BE = BLK * CH
NSLOT = 5


def _zero_acc(zb, acc, s):
    def zrow(i, _):
        zb[i, :] = jnp.zeros((16,), F32)
        return 0

    lax.fori_loop(0, ZR, zrow, 0)

    def zcopy(j, _):
        pltpu.sync_copy(zb, acc.at[pl.ds(s * RPS + j * ZR, ZR)])
        return 0

    lax.fori_loop(0, RPS // ZR, zcopy, 0)


def _spmm_body(mode, *refs):
    if mode == "deg":
        (dst2_hbm, out_hbm, dstb, rows_v, zb, acc, si0, *sems) = refs
        src_hbm = table_hbm = srcb = None
    else:
        (src_hbm, dst2_hbm, table_hbm, out_hbm,
         srcb, dstb, rows_v, zb, acc, si0, *sems) = refs
    sg = sems[0:NSLOT]
    ss = sems[NSLOT:2 * NSLOT]

    c = lax.axis_index("c")
    s = lax.axis_index("s")
    _zero_acc(zb, acc, s)
    if mode == "deg":
        def orow(i, _):
            for p in range(NSLOT):
                rows_v[p, i, :] = jnp.full((16,), 1.0, F32)
            return 0
        lax.fori_loop(0, CH, orow, 0)
    plsc.subcore_barrier()

    if mode == "wide":
        ew = E // NS
        wbase = s * ew
    else:
        ew = E // (NC * NS)
        wbase = (c * NS + s) * ew
    nb = ew // BE
    wrow = wbase // CH

    def fire_idx(t, pb):
        if mode != "deg":
            pltpu.async_copy(src_hbm.at[pl.ds(wbase + t * BE, BE)],
                             srcb.at[pb], si0)
        pltpu.async_copy(dst2_hbm.at[pl.ds(wrow + t * BLK, BLK)],
                         dstb.at[pb], si0)

    def wait_idx(t, pb):
        if mode != "deg":
            pltpu.make_async_copy(src_hbm.at[pl.ds(wbase + t * BE, BE)],
                                  srcb.at[pb], si0).wait()
        pltpu.make_async_copy(dst2_hbm.at[pl.ds(wrow + t * BLK, BLK)],
                              dstb.at[pb], si0).wait()

    def fire_scatter(jj, p):
        t = jj // BLK
        pltpu.async_copy(rows_v.at[p], acc.at[dstb.at[t % 2, jj % BLK]],
                         ss[p], add=True)

    def wait_scatter(p):
        pltpu.make_async_copy(rows_v.at[p], acc.at[dstb.at[0, 0]],
                              ss[p]).wait()

    fire_idx(0, 0)

    LAG = 3

    def body(g5, _):
        for p in range(NSLOT):
            jj = g5 * NSLOT + p
            t = jj // BLK
            pb = t % 2
            if p == 0:
                @pl.when(g5 % 5 == 0)
                def _():
                    wait_idx(t, pb)

                @pl.when((g5 % 5 == 1) & (t + 1 < nb))
                def _():
                    fire_idx(t + 1, (t + 1) % 2)

            @pl.when(g5 >= 1)
            def _():
                wait_scatter(p)

            if mode == "wide":
                shift = c * N_ACC
                cbase = (jj % BLK) * CH
                for k in range(CH // 16):
                    sl = pl.ds(cbase + k * 16, 16)
                    srcb[pb, sl] = srcb[pb, sl] + shift

            pltpu.async_copy(
                table_hbm.at[srcb.at[pb, pl.ds((jj % BLK) * CH, CH)]],
                rows_v.at[p], sg[p])

            sp = (p - LAG) % NSLOT
            cond = True if p >= LAG else (g5 >= 1)

            @pl.when(cond)
            def _():
                pltpu.make_async_copy(
                    table_hbm.at[srcb.at[0, pl.ds(0, CH)]],
                    rows_v.at[sp], sg[sp]).wait()
                fire_scatter(jj - LAG, sp)
        return 0

    def body_deg(g5, _):
        for p in range(NSLOT):
            jj = g5 * NSLOT + p
            t = jj // BLK
            if p == 0:
                @pl.when(g5 % 5 == 0)
                def _():
                    wait_idx(t, t % 2)

                @pl.when((g5 % 5 == 1) & (t + 1 < nb))
                def _():
                    fire_idx(t + 1, (t + 1) % 2)

            @pl.when(g5 >= 1)
            def _():
                wait_scatter(p)

            fire_scatter(jj, p)
        return 0

    if mode == "deg":
        lax.fori_loop(0, ew // CH // NSLOT, body_deg, 0)
    else:
        lax.fori_loop(0, ew // CH // NSLOT, body, 0)
        nch = ew // CH
        for k in range(nch - 3, nch):
            pltpu.make_async_copy(table_hbm.at[srcb.at[0, pl.ds(0, CH)]],
                                  rows_v.at[k % NSLOT], sg[k % NSLOT]).wait()
            fire_scatter(k, k % NSLOT)
    for p in range(NSLOT):
        wait_scatter(p)

    plsc.subcore_barrier()
    pltpu.sync_copy(acc.at[pl.ds(s * RPS, RPS)],
                    out_hbm.at[c, pl.ds(s * RPS, RPS)])


def _make_spmm(mode):
    scratch = []
    if mode != "deg":
        scratch.append(pltpu.VMEM((2, BE), jnp.int32))
    scratch += [
        pltpu.VMEM((2, BLK, CH), jnp.int32),
        pltpu.VMEM((NSLOT, CH, 16), F32),
        pltpu.VMEM((ZR, 16), F32),
        pltpu.VMEM_SHARED((N_ACC, 16), F32),
    ]
    scratch += [pltpu.SemaphoreType.DMA] * (1 + 2 * NSLOT)
    return pl.kernel(
        functools.partial(_spmm_body, mode),
        out_type=jax.ShapeDtypeStruct((NC, N_ACC, 16), F32),
        mesh=_sc_mesh(),
        scratch_types=scratch,
        compiler_params=pltpu.CompilerParams(use_tc_tiling_on_sc=False),
    )


_sc_spmm_wide = _make_spmm("wide")
_sc_spmm_narrow = _make_spmm("narrow")
_sc_deg = _make_spmm("deg")


R8 = 1600
NR8 = N_ACC * 16 // 128
GRID8 = NR8 // R8


def _rs():
    return pl.BlockSpec((R8, 128), lambda i: (i, 0))


def _rs2():
    return pl.BlockSpec((NC, R8, 128), lambda i: (0, i, 0))


def _fs():
    return pl.BlockSpec((128, 128), lambda i: (0, 0))


def _bs():
    return pl.BlockSpec((128,), lambda i: (0,))


def _tc_call(body, in_specs, out_specs, out_shapes):
    return pl.pallas_call(
        body,
        grid=(GRID8,),
        in_specs=in_specs,
        out_specs=out_specs,
        out_shape=out_shapes,
    )


def _dot(a, b):
    return jnp.dot(a, b, preferred_element_type=F32)


def _tck0_body(deg_ref, x_ref, wa_ref, wb_ref, s_ref, g1_ref):
    d = deg_ref[0] + deg_ref[1] + 1.0
    sb = lax.rsqrt(d)
    s_ref[...] = sb
    x8 = x_ref[...]
    g1_ref[0] = sb * _dot(x8, wa_ref[...])
    g1_ref[1] = sb * _dot(x8, wb_ref[...])


def _tck1_body(y_ref, g_ref, s_ref, ba_ref, bb_ref, wa_ref, wb_ref,
               xl1_ref, g2_ref):
    sb = s_ref[...]
    xa = jnp.maximum(sb * (y_ref[0] + g_ref[0]) + ba_ref[...][None, :], 0.0)
    xb = jnp.maximum(sb * (y_ref[1] + g_ref[1]) + bb_ref[...][None, :], 0.0)
    xl1_ref[0] = xa
    xl1_ref[1] = xb
    g2_ref[...] = sb * (_dot(xa, wa_ref[...]) + _dot(xb, wb_ref[...]))


def _tck2_body(y_ref, g_ref, s_ref, b_ref, wa_ref, wb_ref, g3_ref):
    sb = s_ref[...]
    x2 = jnp.maximum(sb * (y_ref[0] + y_ref[1] + g_ref[...])
                     + b_ref[...][None, :], 0.0)
    g3_ref[0] = sb * _dot(x2, wa_ref[...])
    g3_ref[1] = sb * _dot(x2, wb_ref[...])


def _tck3_body(y_ref, g_ref, s_ref, ba_ref, bb_ref, xl1_ref, wa_ref, wb_ref,
               g4_ref):
    sb = s_ref[...]
    ta = jnp.maximum(sb * (y_ref[0] + g_ref[0]) + ba_ref[...][None, :],
                     0.0) + xl1_ref[0]
    tb = jnp.maximum(sb * (y_ref[1] + g_ref[1]) + bb_ref[...][None, :],
                     0.0) + xl1_ref[1]
    g4_ref[...] = sb * (_dot(ta, wa_ref[...]) + _dot(tb, wb_ref[...]))


def _tck4_body(y_ref, g_ref, s_ref, b_ref, x_ref, out_ref):
    sb = s_ref[...]
    z = sb * (y_ref[0] + y_ref[1] + g_ref[...]) + b_ref[...][None, :]
    out_ref[...] = jnp.maximum(z, 0.0) + x_ref[...]


def _p2(shape=(NR8, 128)):
    return jax.ShapeDtypeStruct(shape, F32)


def kernel(x, edge_index, W1, b1, W2, b2, W3, b3, W4, b4):
    ei = edge_index.astype(jnp.int32)
    src = ei[0]
    dst = ei[1]
    dst2 = dst.reshape(E // CH, CH)

    eye8 = jnp.eye(8, dtype=F32)
    w1a = jnp.kron(eye8, W1[:, :16])
    w1b = jnp.kron(eye8, W1[:, 16:])
    w2a = jnp.kron(eye8, W2[:16, :])
    w2b = jnp.kron(eye8, W2[16:, :])
    w3a = jnp.kron(eye8, W3[:, :16])
    w3b = jnp.kron(eye8, W3[:, 16:])
    w4a = jnp.kron(eye8, W4[:16, :])
    w4b = jnp.kron(eye8, W4[16:, :])
    b1a, b1b = jnp.tile(b1[:16], 8), jnp.tile(b1[16:], 8)
    b2t = jnp.tile(b2, 8)
    b3a, b3b = jnp.tile(b3[:16], 8), jnp.tile(b3[16:], 8)
    b4t = jnp.tile(b4, 8)
    x8 = jnp.pad(x.reshape(N * 16 // 128, 128),
                 ((0, NR8 - N * 16 // 128), (0, 0)))

    deg = _sc_deg(dst2).reshape(NC, NR8, 128)

    s8, g1s = _tc_call(
        _tck0_body,
        [_rs2(), _rs(), _fs(), _fs()],
        [_rs(), _rs2()],
        [_p2(), _p2((NC, NR8, 128))],
    )(deg, x8, w1a, w1b)

    y1 = _sc_spmm_wide(src, dst2, g1s.reshape(NC * N_ACC, 16))

    xl1, g2 = _tc_call(
        _tck1_body,
        [_rs2(), _rs2(), _rs(), _bs(), _bs(), _fs(), _fs()],
        [_rs2(), _rs()],
        [_p2((NC, NR8, 128)), _p2()],
    )(y1.reshape(NC, NR8, 128), g1s, s8, b1a, b1b, w2a, w2b)

    y2 = _sc_spmm_narrow(src, dst2, g2.reshape(N_ACC, 16))

    (g3s,) = _tc_call(
        _tck2_body,
        [_rs2(), _rs(), _rs(), _bs(), _fs(), _fs()],
        [_rs2()],
        [_p2((NC, NR8, 128))],
    )(y2.reshape(NC, NR8, 128), g2, s8, b2t, w3a, w3b)

    y3 = _sc_spmm_wide(src, dst2, g3s.reshape(NC * N_ACC, 16))

    (g4,) = _tc_call(
        _tck3_body,
        [_rs2(), _rs2(), _rs(), _bs(), _bs(), _rs2(), _fs(), _fs()],
        [_rs()],
        [_p2()],
    )(y3.reshape(NC, NR8, 128), g3s, s8, b3a, b3b, xl1, w4a, w4b)

    y4 = _sc_spmm_narrow(src, dst2, g4.reshape(N_ACC, 16))

    (out8,) = _tc_call(
        _tck4_body,
        [_rs2(), _rs(), _rs(), _bs(), _rs()],
        [_rs()],
        [_p2()],
    )(y4.reshape(NC, NR8, 128), g4, s8, b4t, x8)

    return out8.reshape(N_ACC, 16)[:N]


# speedup vs baseline: 68.3502x; 1.0654x over previous
import functools

import jax
import jax.numpy as jnp
from jax import lax
from jax.experimental import pallas as pl
from jax.experimental.pallas import tpu as pltpu
from jax.experimental.pallas import tpu_sc as plsc

N = 100000
E = 3200000
NC = 2
NS = 16
CH = 80
N_ACC = 102400
RPS = N_ACC // NS
ZR = 640

F32 = jnp.float32


def _sc_mesh():
    return plsc.VectorSubcoreMesh(core_axis_name="c", subcore_axis_name="s",
                                  num_cores=NC, num_subcores=NS)


BLK = 25
BE = BLK * CH
NSLOT = 5


def _zero_acc(zb, acc, s):
    def zrow(i, _):
        zb[i, :] = jnp.zeros((16,), F32)
        return 0

    lax.fori_loop(0, ZR, zrow, 0)

    def zcopy(j, _):
        pltpu.sync_copy(zb, acc.at[pl.ds(s * RPS + j * ZR, ZR)])
        return 0

    lax.fori_loop(0, RPS // ZR, zcopy, 0)


def _spmm_body(mode, *refs):
    if mode == "deg":
        (dst2_hbm, out_hbm, dstb, rows_v, zb, acc, si0, *sems) = refs
        src_hbm = table_hbm = srcb = None
    else:
        (src_hbm, dst2_hbm, table_hbm, out_hbm,
         srcb, dstb, rows_v, zb, acc, si0, *sems) = refs
    sg = sems[0:NSLOT]
    ss = sems[NSLOT:2 * NSLOT]

    c = lax.axis_index("c")
    s = lax.axis_index("s")
    _zero_acc(zb, acc, s)
    if mode == "deg":
        def orow(i, _):
            for p in range(NSLOT):
                rows_v[p, i, :] = jnp.full((16,), 1.0, F32)
            return 0
        lax.fori_loop(0, CH, orow, 0)
    plsc.subcore_barrier()

    if mode == "wide":
        ew = E // NS
        wbase = s * ew
    else:
        ew = E // (NC * NS)
        wbase = (c * NS + s) * ew
    nb = ew // BE
    wrow = wbase // CH

    def fire_idx(t, pb):
        if mode != "deg":
            pltpu.async_copy(src_hbm.at[pl.ds(wbase + t * BE, BE)],
                             srcb.at[pb], si0)
        pltpu.async_copy(dst2_hbm.at[pl.ds(wrow + t * BLK, BLK)],
                         dstb.at[pb], si0)

    def wait_idx(t, pb):
        if mode != "deg":
            pltpu.make_async_copy(src_hbm.at[pl.ds(wbase + t * BE, BE)],
                                  srcb.at[pb], si0).wait()
        pltpu.make_async_copy(dst2_hbm.at[pl.ds(wrow + t * BLK, BLK)],
                              dstb.at[pb], si0).wait()

    def fire_scatter(jj, p):
        t = jj // BLK
        pltpu.async_copy(rows_v.at[p], acc.at[dstb.at[t % 2, jj % BLK]],
                         ss[p], add=True)

    def wait_scatter(p):
        pltpu.make_async_copy(rows_v.at[p], acc.at[dstb.at[0, 0]],
                              ss[p]).wait()

    fire_idx(0, 0)

    LAG = 4

    def body(g5, _):
        for p in range(NSLOT):
            jj = g5 * NSLOT + p
            t = jj // BLK
            pb = t % 2
            if p == 0:
                @pl.when(g5 % 5 == 0)
                def _():
                    wait_idx(t, pb)

                @pl.when((g5 % 5 == 1) & (t + 1 < nb))
                def _():
                    fire_idx(t + 1, (t + 1) % 2)

            @pl.when(g5 >= 1)
            def _():
                wait_scatter(p)

            if mode == "wide":
                shift = c * N_ACC
                cbase = (jj % BLK) * CH
                for k in range(CH // 16):
                    sl = pl.ds(cbase + k * 16, 16)
                    srcb[pb, sl] = srcb[pb, sl] + shift

            pltpu.async_copy(
                table_hbm.at[srcb.at[pb, pl.ds((jj % BLK) * CH, CH)]],
                rows_v.at[p], sg[p])

            sp = (p - LAG) % NSLOT
            cond = True if p >= LAG else (g5 >= 1)

            @pl.when(cond)
            def _():
                pltpu.make_async_copy(
                    table_hbm.at[srcb.at[0, pl.ds(0, CH)]],
                    rows_v.at[sp], sg[sp]).wait()
                fire_scatter(jj - LAG, sp)
        return 0

    def body_deg(g5, _):
        for p in range(NSLOT):
            jj = g5 * NSLOT + p
            t = jj // BLK
            if p == 0:
                @pl.when(g5 % 5 == 0)
                def _():
                    wait_idx(t, t % 2)

                @pl.when((g5 % 5 == 1) & (t + 1 < nb))
                def _():
                    fire_idx(t + 1, (t + 1) % 2)

            @pl.when(g5 >= 1)
            def _():
                wait_scatter(p)

            fire_scatter(jj, p)
        return 0

    if mode == "deg":
        lax.fori_loop(0, ew // CH // NSLOT, body_deg, 0)
    else:
        lax.fori_loop(0, ew // CH // NSLOT, body, 0)
        nch = ew // CH
        for k in range(nch - 4, nch):
            pltpu.make_async_copy(table_hbm.at[srcb.at[0, pl.ds(0, CH)]],
                                  rows_v.at[k % NSLOT], sg[k % NSLOT]).wait()
            fire_scatter(k, k % NSLOT)
    for p in range(NSLOT):
        wait_scatter(p)

    plsc.subcore_barrier()
    pltpu.sync_copy(acc.at[pl.ds(s * RPS, RPS)],
                    out_hbm.at[c, pl.ds(s * RPS, RPS)])


def _make_spmm(mode):
    scratch = []
    if mode != "deg":
        scratch.append(pltpu.VMEM((2, BE), jnp.int32))
    scratch += [
        pltpu.VMEM((2, BLK, CH), jnp.int32),
        pltpu.VMEM((NSLOT, CH, 16), F32),
        pltpu.VMEM((ZR, 16), F32),
        pltpu.VMEM_SHARED((N_ACC, 16), F32),
    ]
    scratch += [pltpu.SemaphoreType.DMA] * (1 + 2 * NSLOT)
    return pl.kernel(
        functools.partial(_spmm_body, mode),
        out_type=jax.ShapeDtypeStruct((NC, N_ACC, 16), F32),
        mesh=_sc_mesh(),
        scratch_types=scratch,
        compiler_params=pltpu.CompilerParams(use_tc_tiling_on_sc=False),
    )


_sc_spmm_wide = _make_spmm("wide")
_sc_spmm_narrow = _make_spmm("narrow")
_sc_deg = _make_spmm("deg")


R8 = 1600
NR8 = N_ACC * 16 // 128
GRID8 = NR8 // R8


def _rs():
    return pl.BlockSpec((R8, 128), lambda i: (i, 0))


def _rs2():
    return pl.BlockSpec((NC, R8, 128), lambda i: (0, i, 0))


def _fs():
    return pl.BlockSpec((128, 128), lambda i: (0, 0))


def _bs():
    return pl.BlockSpec((128,), lambda i: (0,))


def _tc_call(body, in_specs, out_specs, out_shapes):
    return pl.pallas_call(
        body,
        grid=(GRID8,),
        in_specs=in_specs,
        out_specs=out_specs,
        out_shape=out_shapes,
    )


def _dot(a, b):
    return jnp.dot(a, b, preferred_element_type=F32)


def _tck0_body(deg_ref, x_ref, wa_ref, wb_ref, s_ref, g1_ref):
    d = deg_ref[0] + deg_ref[1] + 1.0
    sb = lax.rsqrt(d)
    s_ref[...] = sb
    x8 = x_ref[...]
    g1_ref[0] = sb * _dot(x8, wa_ref[...])
    g1_ref[1] = sb * _dot(x8, wb_ref[...])


def _tck1_body(y_ref, g_ref, s_ref, ba_ref, bb_ref, wa_ref, wb_ref,
               xl1_ref, g2_ref):
    sb = s_ref[...]
    xa = jnp.maximum(sb * (y_ref[0] + g_ref[0]) + ba_ref[...][None, :], 0.0)
    xb = jnp.maximum(sb * (y_ref[1] + g_ref[1]) + bb_ref[...][None, :], 0.0)
    xl1_ref[0] = xa
    xl1_ref[1] = xb
    g2_ref[...] = sb * (_dot(xa, wa_ref[...]) + _dot(xb, wb_ref[...]))


def _tck2_body(y_ref, g_ref, s_ref, b_ref, wa_ref, wb_ref, g3_ref):
    sb = s_ref[...]
    x2 = jnp.maximum(sb * (y_ref[0] + y_ref[1] + g_ref[...])
                     + b_ref[...][None, :], 0.0)
    g3_ref[0] = sb * _dot(x2, wa_ref[...])
    g3_ref[1] = sb * _dot(x2, wb_ref[...])


def _tck3_body(y_ref, g_ref, s_ref, ba_ref, bb_ref, xl1_ref, wa_ref, wb_ref,
               g4_ref):
    sb = s_ref[...]
    ta = jnp.maximum(sb * (y_ref[0] + g_ref[0]) + ba_ref[...][None, :],
                     0.0) + xl1_ref[0]
    tb = jnp.maximum(sb * (y_ref[1] + g_ref[1]) + bb_ref[...][None, :],
                     0.0) + xl1_ref[1]
    g4_ref[...] = sb * (_dot(ta, wa_ref[...]) + _dot(tb, wb_ref[...]))


def _tck4_body(y_ref, g_ref, s_ref, b_ref, x_ref, out_ref):
    sb = s_ref[...]
    z = sb * (y_ref[0] + y_ref[1] + g_ref[...]) + b_ref[...][None, :]
    out_ref[...] = jnp.maximum(z, 0.0) + x_ref[...]


def _p2(shape=(NR8, 128)):
    return jax.ShapeDtypeStruct(shape, F32)


def kernel(x, edge_index, W1, b1, W2, b2, W3, b3, W4, b4):
    ei = edge_index.astype(jnp.int32)
    src = ei[0]
    dst = ei[1]
    dst2 = dst.reshape(E // CH, CH)

    eye8 = jnp.eye(8, dtype=F32)
    w1a = jnp.kron(eye8, W1[:, :16])
    w1b = jnp.kron(eye8, W1[:, 16:])
    w2a = jnp.kron(eye8, W2[:16, :])
    w2b = jnp.kron(eye8, W2[16:, :])
    w3a = jnp.kron(eye8, W3[:, :16])
    w3b = jnp.kron(eye8, W3[:, 16:])
    w4a = jnp.kron(eye8, W4[:16, :])
    w4b = jnp.kron(eye8, W4[16:, :])
    b1a, b1b = jnp.tile(b1[:16], 8), jnp.tile(b1[16:], 8)
    b2t = jnp.tile(b2, 8)
    b3a, b3b = jnp.tile(b3[:16], 8), jnp.tile(b3[16:], 8)
    b4t = jnp.tile(b4, 8)
    x8 = jnp.pad(x.reshape(N * 16 // 128, 128),
                 ((0, NR8 - N * 16 // 128), (0, 0)))

    deg = _sc_deg(dst2).reshape(NC, NR8, 128)

    s8, g1s = _tc_call(
        _tck0_body,
        [_rs2(), _rs(), _fs(), _fs()],
        [_rs(), _rs2()],
        [_p2(), _p2((NC, NR8, 128))],
    )(deg, x8, w1a, w1b)

    y1 = _sc_spmm_wide(src, dst2, g1s.reshape(NC * N_ACC, 16))

    xl1, g2 = _tc_call(
        _tck1_body,
        [_rs2(), _rs2(), _rs(), _bs(), _bs(), _fs(), _fs()],
        [_rs2(), _rs()],
        [_p2((NC, NR8, 128)), _p2()],
    )(y1.reshape(NC, NR8, 128), g1s, s8, b1a, b1b, w2a, w2b)

    y2 = _sc_spmm_narrow(src, dst2, g2.reshape(N_ACC, 16))

    (g3s,) = _tc_call(
        _tck2_body,
        [_rs2(), _rs(), _rs(), _bs(), _fs(), _fs()],
        [_rs2()],
        [_p2((NC, NR8, 128))],
    )(y2.reshape(NC, NR8, 128), g2, s8, b2t, w3a, w3b)

    y3 = _sc_spmm_wide(src, dst2, g3s.reshape(NC * N_ACC, 16))

    (g4,) = _tc_call(
        _tck3_body,
        [_rs2(), _rs2(), _rs(), _bs(), _bs(), _rs2(), _fs(), _fs()],
        [_rs()],
        [_p2()],
    )(y3.reshape(NC, NR8, 128), g3s, s8, b3a, b3b, xl1, w4a, w4b)

    y4 = _sc_spmm_narrow(src, dst2, g4.reshape(N_ACC, 16))

    (out8,) = _tc_call(
        _tck4_body,
        [_rs2(), _rs(), _rs(), _bs(), _rs()],
        [_rs()],
        [_p2()],
    )(y4.reshape(NC, NR8, 128), g4, s8, b4t, x8)

    return out8.reshape(N_ACC, 16)[:N]


# speedup vs baseline: 68.4887x; 1.0020x over previous
import functools

import jax
import jax.numpy as jnp
from jax import lax
from jax.experimental import pallas as pl
from jax.experimental.pallas import tpu as pltpu
from jax.experimental.pallas import tpu_sc as plsc

N = 100000
E = 3200000
NC = 2
NS = 16
CH = 80
N_ACC = 102400
RPS = N_ACC // NS
ZR = 640

F32 = jnp.float32


def _sc_mesh():
    return plsc.VectorSubcoreMesh(core_axis_name="c", subcore_axis_name="s",
                                  num_cores=NC, num_subcores=NS)


BLK = 25
BE = BLK * CH
NSLOT = 5


def _zero_acc(zb, acc, s, sem):
    def zrow(i, _):
        zb[i, :] = jnp.zeros((16,), F32)
        return 0

    lax.fori_loop(0, ZR, zrow, 0)

    def zcopy(j, _):
        pltpu.async_copy(zb, acc.at[pl.ds(s * RPS + j * ZR, ZR)], sem)
        return 0

    lax.fori_loop(0, RPS // ZR, zcopy, 0)

    def zwait(j, _):
        pltpu.make_async_copy(zb, acc.at[pl.ds(s * RPS + j * ZR, ZR)],
                              sem).wait()
        return 0

    lax.fori_loop(0, RPS // ZR, zwait, 0)


def _spmm_body(mode, *refs):
    if mode == "deg":
        (dst2_hbm, out_hbm, dstb, rows_v, zb, acc, si0, *sems) = refs
        src_hbm = table_hbm = srcb = None
    else:
        (src_hbm, dst2_hbm, table_hbm, out_hbm,
         srcb, dstb, rows_v, zb, acc, si0, *sems) = refs
    sg = sems[0:NSLOT]
    ss = sems[NSLOT:2 * NSLOT]

    c = lax.axis_index("c")
    s = lax.axis_index("s")
    _zero_acc(zb, acc, s, sems[0])
    if mode == "deg":
        def orow(i, _):
            for p in range(NSLOT):
                rows_v[p, i, :] = jnp.full((16,), 1.0, F32)
            return 0
        lax.fori_loop(0, CH, orow, 0)
    plsc.subcore_barrier()

    if mode == "wide":
        ew = E // NS
        wbase = s * ew
    else:
        ew = E // (NC * NS)
        wbase = (c * NS + s) * ew
    nb = ew // BE
    wrow = wbase // CH

    def fire_idx(t, pb):
        if mode != "deg":
            pltpu.async_copy(src_hbm.at[pl.ds(wbase + t * BE, BE)],
                             srcb.at[pb], si0)
        pltpu.async_copy(dst2_hbm.at[pl.ds(wrow + t * BLK, BLK)],
                         dstb.at[pb], si0)

    def wait_idx(t, pb):
        if mode != "deg":
            pltpu.make_async_copy(src_hbm.at[pl.ds(wbase + t * BE, BE)],
                                  srcb.at[pb], si0).wait()
        pltpu.make_async_copy(dst2_hbm.at[pl.ds(wrow + t * BLK, BLK)],
                              dstb.at[pb], si0).wait()

    def fire_scatter(jj, p):
        t = jj // BLK
        pltpu.async_copy(rows_v.at[p], acc.at[dstb.at[t % 2, jj % BLK]],
                         ss[p], add=True)

    def wait_scatter(p):
        pltpu.make_async_copy(rows_v.at[p], acc.at[dstb.at[0, 0]],
                              ss[p]).wait()

    fire_idx(0, 0)

    LAG = 4

    def body(g5, _):
        for p in range(NSLOT):
            jj = g5 * NSLOT + p
            t = jj // BLK
            pb = t % 2
            if p == 0:
                @pl.when(g5 % 5 == 0)
                def _():
                    wait_idx(t, pb)

                @pl.when((g5 % 5 == 1) & (t + 1 < nb))
                def _():
                    fire_idx(t + 1, (t + 1) % 2)

            @pl.when(g5 >= 1)
            def _():
                wait_scatter(p)

            if mode == "wide":
                shift = c * N_ACC
                cbase = (jj % BLK) * CH
                for k in range(CH // 16):
                    sl = pl.ds(cbase + k * 16, 16)
                    srcb[pb, sl] = srcb[pb, sl] + shift

            pltpu.async_copy(
                table_hbm.at[srcb.at[pb, pl.ds((jj % BLK) * CH, CH)]],
                rows_v.at[p], sg[p])

            sp = (p - LAG) % NSLOT
            cond = True if p >= LAG else (g5 >= 1)

            @pl.when(cond)
            def _():
                pltpu.make_async_copy(
                    table_hbm.at[srcb.at[0, pl.ds(0, CH)]],
                    rows_v.at[sp], sg[sp]).wait()
                fire_scatter(jj - LAG, sp)
        return 0

    def body_deg(g5, _):
        for p in range(NSLOT):
            jj = g5 * NSLOT + p
            t = jj // BLK
            if p == 0:
                @pl.when(g5 % 5 == 0)
                def _():
                    wait_idx(t, t % 2)

                @pl.when((g5 % 5 == 1) & (t + 1 < nb))
                def _():
                    fire_idx(t + 1, (t + 1) % 2)

            @pl.when(g5 >= 1)
            def _():
                wait_scatter(p)

            fire_scatter(jj, p)
        return 0

    if mode == "deg":
        lax.fori_loop(0, ew // CH // NSLOT, body_deg, 0)
    else:
        lax.fori_loop(0, ew // CH // NSLOT, body, 0)
        nch = ew // CH
        for k in range(nch - 4, nch):
            pltpu.make_async_copy(table_hbm.at[srcb.at[0, pl.ds(0, CH)]],
                                  rows_v.at[k % NSLOT], sg[k % NSLOT]).wait()
            fire_scatter(k, k % NSLOT)
    for p in range(NSLOT):
        wait_scatter(p)

    plsc.subcore_barrier()
    pltpu.sync_copy(acc.at[pl.ds(s * RPS, RPS)],
                    out_hbm.at[c, pl.ds(s * RPS, RPS)])


def _make_spmm(mode):
    scratch = []
    if mode != "deg":
        scratch.append(pltpu.VMEM((2, BE), jnp.int32))
    scratch += [
        pltpu.VMEM((2, BLK, CH), jnp.int32),
        pltpu.VMEM((NSLOT, CH, 16), F32),
        pltpu.VMEM((ZR, 16), F32),
        pltpu.VMEM_SHARED((N_ACC, 16), F32),
    ]
    scratch += [pltpu.SemaphoreType.DMA] * (1 + 2 * NSLOT)
    return pl.kernel(
        functools.partial(_spmm_body, mode),
        out_type=jax.ShapeDtypeStruct((NC, N_ACC, 16), F32),
        mesh=_sc_mesh(),
        scratch_types=scratch,
        compiler_params=pltpu.CompilerParams(use_tc_tiling_on_sc=False),
    )


_sc_spmm_wide = _make_spmm("wide")
_sc_spmm_narrow = _make_spmm("narrow")
_sc_deg = _make_spmm("deg")


R8 = 1600
NR8 = N_ACC * 16 // 128
GRID8 = NR8 // R8


def _rs():
    return pl.BlockSpec((R8, 128), lambda i: (i, 0))


def _rs2():
    return pl.BlockSpec((NC, R8, 128), lambda i: (0, i, 0))


def _fs():
    return pl.BlockSpec((128, 128), lambda i: (0, 0))


def _bs():
    return pl.BlockSpec((128,), lambda i: (0,))


def _tc_call(body, in_specs, out_specs, out_shapes):
    return pl.pallas_call(
        body,
        grid=(GRID8,),
        in_specs=in_specs,
        out_specs=out_specs,
        out_shape=out_shapes,
    )


def _dot(a, b):
    return jnp.dot(a, b, preferred_element_type=F32)


def _tck0_body(deg_ref, x_ref, wa_ref, wb_ref, s_ref, g1_ref):
    d = deg_ref[0] + deg_ref[1] + 1.0
    sb = lax.rsqrt(d)
    s_ref[...] = sb
    x8 = x_ref[...]
    g1_ref[0] = sb * _dot(x8, wa_ref[...])
    g1_ref[1] = sb * _dot(x8, wb_ref[...])


def _tck1_body(y_ref, g_ref, s_ref, ba_ref, bb_ref, wa_ref, wb_ref,
               xl1_ref, g2_ref):
    sb = s_ref[...]
    xa = jnp.maximum(sb * (y_ref[0] + g_ref[0]) + ba_ref[...][None, :], 0.0)
    xb = jnp.maximum(sb * (y_ref[1] + g_ref[1]) + bb_ref[...][None, :], 0.0)
    xl1_ref[0] = xa
    xl1_ref[1] = xb
    g2_ref[...] = sb * (_dot(xa, wa_ref[...]) + _dot(xb, wb_ref[...]))


def _tck2_body(y_ref, g_ref, s_ref, b_ref, wa_ref, wb_ref, g3_ref):
    sb = s_ref[...]
    x2 = jnp.maximum(sb * (y_ref[0] + y_ref[1] + g_ref[...])
                     + b_ref[...][None, :], 0.0)
    g3_ref[0] = sb * _dot(x2, wa_ref[...])
    g3_ref[1] = sb * _dot(x2, wb_ref[...])


def _tck3_body(y_ref, g_ref, s_ref, ba_ref, bb_ref, xl1_ref, wa_ref, wb_ref,
               g4_ref):
    sb = s_ref[...]
    ta = jnp.maximum(sb * (y_ref[0] + g_ref[0]) + ba_ref[...][None, :],
                     0.0) + xl1_ref[0]
    tb = jnp.maximum(sb * (y_ref[1] + g_ref[1]) + bb_ref[...][None, :],
                     0.0) + xl1_ref[1]
    g4_ref[...] = sb * (_dot(ta, wa_ref[...]) + _dot(tb, wb_ref[...]))


def _tck4_body(y_ref, g_ref, s_ref, b_ref, x_ref, out_ref):
    sb = s_ref[...]
    z = sb * (y_ref[0] + y_ref[1] + g_ref[...]) + b_ref[...][None, :]
    out_ref[...] = jnp.maximum(z, 0.0) + x_ref[...]


def _p2(shape=(NR8, 128)):
    return jax.ShapeDtypeStruct(shape, F32)


def kernel(x, edge_index, W1, b1, W2, b2, W3, b3, W4, b4):
    ei = edge_index.astype(jnp.int32)
    src = ei[0]
    dst = ei[1]
    dst2 = dst.reshape(E // CH, CH)

    eye8 = jnp.eye(8, dtype=F32)
    w1a = jnp.kron(eye8, W1[:, :16])
    w1b = jnp.kron(eye8, W1[:, 16:])
    w2a = jnp.kron(eye8, W2[:16, :])
    w2b = jnp.kron(eye8, W2[16:, :])
    w3a = jnp.kron(eye8, W3[:, :16])
    w3b = jnp.kron(eye8, W3[:, 16:])
    w4a = jnp.kron(eye8, W4[:16, :])
    w4b = jnp.kron(eye8, W4[16:, :])
    b1a, b1b = jnp.tile(b1[:16], 8), jnp.tile(b1[16:], 8)
    b2t = jnp.tile(b2, 8)
    b3a, b3b = jnp.tile(b3[:16], 8), jnp.tile(b3[16:], 8)
    b4t = jnp.tile(b4, 8)
    x8 = jnp.pad(x.reshape(N * 16 // 128, 128),
                 ((0, NR8 - N * 16 // 128), (0, 0)))

    deg = _sc_deg(dst2).reshape(NC, NR8, 128)

    s8, g1s = _tc_call(
        _tck0_body,
        [_rs2(), _rs(), _fs(), _fs()],
        [_rs(), _rs2()],
        [_p2(), _p2((NC, NR8, 128))],
    )(deg, x8, w1a, w1b)

    y1 = _sc_spmm_wide(src, dst2, g1s.reshape(NC * N_ACC, 16))

    xl1, g2 = _tc_call(
        _tck1_body,
        [_rs2(), _rs2(), _rs(), _bs(), _bs(), _fs(), _fs()],
        [_rs2(), _rs()],
        [_p2((NC, NR8, 128)), _p2()],
    )(y1.reshape(NC, NR8, 128), g1s, s8, b1a, b1b, w2a, w2b)

    y2 = _sc_spmm_narrow(src, dst2, g2.reshape(N_ACC, 16))

    (g3s,) = _tc_call(
        _tck2_body,
        [_rs2(), _rs(), _rs(), _bs(), _fs(), _fs()],
        [_rs2()],
        [_p2((NC, NR8, 128))],
    )(y2.reshape(NC, NR8, 128), g2, s8, b2t, w3a, w3b)

    y3 = _sc_spmm_wide(src, dst2, g3s.reshape(NC * N_ACC, 16))

    (g4,) = _tc_call(
        _tck3_body,
        [_rs2(), _rs2(), _rs(), _bs(), _bs(), _rs2(), _fs(), _fs()],
        [_rs()],
        [_p2()],
    )(y3.reshape(NC, NR8, 128), g3s, s8, b3a, b3b, xl1, w4a, w4b)

    y4 = _sc_spmm_narrow(src, dst2, g4.reshape(N_ACC, 16))

    (out8,) = _tc_call(
        _tck4_body,
        [_rs2(), _rs(), _rs(), _bs(), _rs()],
        [_rs()],
        [_p2()],
    )(y4.reshape(NC, NR8, 128), g4, s8, b4t, x8)

    return out8.reshape(N_ACC, 16)[:N]


# speedup vs baseline: 85.1463x; 1.2432x over previous
import functools

import jax
import jax.numpy as jnp
from jax import lax
from jax.experimental import pallas as pl
from jax.experimental.pallas import tpu as pltpu
from jax.experimental.pallas import tpu_sc as plsc

N = 100000
E = 3200000
E_PAD = 3276800
NC = 2
NS = 16
CH = 128
N_ACC = 102400
RPS = N_ACC // NS
ZR = 160

F32 = jnp.float32


def _sc_mesh():
    return plsc.VectorSubcoreMesh(core_axis_name="c", subcore_axis_name="s",
                                  num_cores=NC, num_subcores=NS)


BLK = 25
BE = BLK * CH
NSLOT = 5


def _zero_acc(zb, acc, s, sem):
    def zrow(i, _):
        zb[i, :] = jnp.zeros((16,), F32)
        return 0

    lax.fori_loop(0, ZR, zrow, 0)

    def zcopy(j, _):
        pltpu.async_copy(zb, acc.at[pl.ds(s * RPS + j * ZR, ZR)], sem)
        return 0

    lax.fori_loop(0, RPS // ZR, zcopy, 0)

    def zwait(j, _):
        pltpu.make_async_copy(zb, acc.at[pl.ds(s * RPS + j * ZR, ZR)],
                              sem).wait()
        return 0

    lax.fori_loop(0, RPS // ZR, zwait, 0)


def _spmm_body(mode, *refs):
    if mode == "deg":
        (dst2_hbm, out_hbm, dstb, rows_v, zb, acc, si0, *sems) = refs
        src_hbm = table_hbm = srcb = None
    else:
        (src_hbm, dst2_hbm, table_hbm, out_hbm,
         srcb, dstb, rows_v, zb, acc, si0, *sems) = refs
    sg = sems[0:NSLOT]
    ss = sems[NSLOT:2 * NSLOT]

    c = lax.axis_index("c")
    s = lax.axis_index("s")
    _zero_acc(zb, acc, s, sems[0])
    if mode == "deg":
        def orow(i, _):
            for p in range(NSLOT):
                rows_v[p, i, :] = jnp.full((16,), 1.0, F32)
            return 0
        lax.fori_loop(0, CH, orow, 0)
    plsc.subcore_barrier()

    if mode == "wide":
        ew = E_PAD // NS
        wbase = s * ew
    else:
        ew = E_PAD // (NC * NS)
        wbase = (c * NS + s) * ew
    nb = ew // BE
    wrow = wbase // CH

    def fire_idx(t, pb):
        if mode != "deg":
            pltpu.async_copy(src_hbm.at[pl.ds(wbase + t * BE, BE)],
                             srcb.at[pb], si0)
        pltpu.async_copy(dst2_hbm.at[pl.ds(wrow + t * BLK, BLK)],
                         dstb.at[pb], si0)

    def wait_idx(t, pb):
        if mode != "deg":
            pltpu.make_async_copy(src_hbm.at[pl.ds(wbase + t * BE, BE)],
                                  srcb.at[pb], si0).wait()
        pltpu.make_async_copy(dst2_hbm.at[pl.ds(wrow + t * BLK, BLK)],
                              dstb.at[pb], si0).wait()

    def fire_scatter(jj, p):
        t = jj // BLK
        pltpu.async_copy(rows_v.at[p], acc.at[dstb.at[t % 2, jj % BLK]],
                         ss[p], add=True)

    def wait_scatter(p):
        pltpu.make_async_copy(rows_v.at[p], acc.at[dstb.at[0, 0]],
                              ss[p]).wait()

    fire_idx(0, 0)

    LAG = 4

    def body(g5, _):
        for p in range(NSLOT):
            jj = g5 * NSLOT + p
            t = jj // BLK
            pb = t % 2
            if p == 0:
                @pl.when(g5 % 5 == 0)
                def _():
                    wait_idx(t, pb)

                @pl.when((g5 % 5 == 1) & (t + 1 < nb))
                def _():
                    fire_idx(t + 1, (t + 1) % 2)

            @pl.when(g5 >= 1)
            def _():
                wait_scatter(p)

            if mode == "wide":
                shift = c * N_ACC
                cbase = (jj % BLK) * CH
                for k in range(CH // 16):
                    sl = pl.ds(cbase + k * 16, 16)
                    srcb[pb, sl] = srcb[pb, sl] + shift

            pltpu.async_copy(
                table_hbm.at[srcb.at[pb, pl.ds((jj % BLK) * CH, CH)]],
                rows_v.at[p], sg[p])

            sp = (p - LAG) % NSLOT
            cond = True if p >= LAG else (g5 >= 1)

            @pl.when(cond)
            def _():
                pltpu.make_async_copy(
                    table_hbm.at[srcb.at[0, pl.ds(0, CH)]],
                    rows_v.at[sp], sg[sp]).wait()
                fire_scatter(jj - LAG, sp)
        return 0

    def body_deg(g5, _):
        for p in range(NSLOT):
            jj = g5 * NSLOT + p
            t = jj // BLK
            if p == 0:
                @pl.when(g5 % 5 == 0)
                def _():
                    wait_idx(t, t % 2)

                @pl.when((g5 % 5 == 1) & (t + 1 < nb))
                def _():
                    fire_idx(t + 1, (t + 1) % 2)

            @pl.when(g5 >= 1)
            def _():
                wait_scatter(p)

            fire_scatter(jj, p)
        return 0

    if mode == "deg":
        lax.fori_loop(0, ew // CH // NSLOT, body_deg, 0)
    else:
        lax.fori_loop(0, ew // CH // NSLOT, body, 0)
        nch = ew // CH
        for k in range(nch - 4, nch):
            pltpu.make_async_copy(table_hbm.at[srcb.at[0, pl.ds(0, CH)]],
                                  rows_v.at[k % NSLOT], sg[k % NSLOT]).wait()
            fire_scatter(k, k % NSLOT)
    for p in range(NSLOT):
        wait_scatter(p)

    plsc.subcore_barrier()
    pltpu.sync_copy(acc.at[pl.ds(s * RPS, RPS)],
                    out_hbm.at[c, pl.ds(s * RPS, RPS)])


def _make_spmm(mode):
    scratch = []
    if mode != "deg":
        scratch.append(pltpu.VMEM((2, BE), jnp.int32))
    scratch += [
        pltpu.VMEM((2, BLK, CH), jnp.int32),
        pltpu.VMEM((NSLOT, CH, 16), F32),
        pltpu.VMEM((ZR, 16), F32),
        pltpu.VMEM_SHARED((N_ACC, 16), F32),
    ]
    scratch += [pltpu.SemaphoreType.DMA] * (1 + 2 * NSLOT)
    return pl.kernel(
        functools.partial(_spmm_body, mode),
        out_type=jax.ShapeDtypeStruct((NC, N_ACC, 16), F32),
        mesh=_sc_mesh(),
        scratch_types=scratch,
        compiler_params=pltpu.CompilerParams(use_tc_tiling_on_sc=False),
    )


_sc_spmm_wide = _make_spmm("wide")
_sc_spmm_narrow = _make_spmm("narrow")
_sc_deg = _make_spmm("deg")


R8 = 1600
NR8 = N_ACC * 16 // 128
GRID8 = NR8 // R8


def _rs():
    return pl.BlockSpec((R8, 128), lambda i: (i, 0))


def _rs2():
    return pl.BlockSpec((NC, R8, 128), lambda i: (0, i, 0))


def _fs():
    return pl.BlockSpec((128, 128), lambda i: (0, 0))


def _bs():
    return pl.BlockSpec((128,), lambda i: (0,))


def _tc_call(body, in_specs, out_specs, out_shapes):
    return pl.pallas_call(
        body,
        grid=(GRID8,),
        in_specs=in_specs,
        out_specs=out_specs,
        out_shape=out_shapes,
    )


def _dot(a, b):
    return jnp.dot(a, b, preferred_element_type=F32)


def _tck0_body(deg_ref, x_ref, wa_ref, wb_ref, s_ref, g1_ref):
    d = deg_ref[0] + deg_ref[1] + 1.0
    sb = lax.rsqrt(d)
    s_ref[...] = sb
    x8 = x_ref[...]
    g1_ref[0] = sb * _dot(x8, wa_ref[...])
    g1_ref[1] = sb * _dot(x8, wb_ref[...])


def _tck1_body(y_ref, g_ref, s_ref, ba_ref, bb_ref, wa_ref, wb_ref,
               xl1_ref, g2_ref):
    sb = s_ref[...]
    xa = jnp.maximum(sb * (y_ref[0] + g_ref[0]) + ba_ref[...][None, :], 0.0)
    xb = jnp.maximum(sb * (y_ref[1] + g_ref[1]) + bb_ref[...][None, :], 0.0)
    xl1_ref[0] = xa
    xl1_ref[1] = xb
    g2_ref[...] = sb * (_dot(xa, wa_ref[...]) + _dot(xb, wb_ref[...]))


def _tck2_body(y_ref, g_ref, s_ref, b_ref, wa_ref, wb_ref, g3_ref):
    sb = s_ref[...]
    x2 = jnp.maximum(sb * (y_ref[0] + y_ref[1] + g_ref[...])
                     + b_ref[...][None, :], 0.0)
    g3_ref[0] = sb * _dot(x2, wa_ref[...])
    g3_ref[1] = sb * _dot(x2, wb_ref[...])


def _tck3_body(y_ref, g_ref, s_ref, ba_ref, bb_ref, xl1_ref, wa_ref, wb_ref,
               g4_ref):
    sb = s_ref[...]
    ta = jnp.maximum(sb * (y_ref[0] + g_ref[0]) + ba_ref[...][None, :],
                     0.0) + xl1_ref[0]
    tb = jnp.maximum(sb * (y_ref[1] + g_ref[1]) + bb_ref[...][None, :],
                     0.0) + xl1_ref[1]
    g4_ref[...] = sb * (_dot(ta, wa_ref[...]) + _dot(tb, wb_ref[...]))


def _tck4_body(y_ref, g_ref, s_ref, b_ref, x_ref, out_ref):
    sb = s_ref[...]
    z = sb * (y_ref[0] + y_ref[1] + g_ref[...]) + b_ref[...][None, :]
    out_ref[...] = jnp.maximum(z, 0.0) + x_ref[...]


def _p2(shape=(NR8, 128)):
    return jax.ShapeDtypeStruct(shape, F32)


def kernel(x, edge_index, W1, b1, W2, b2, W3, b3, W4, b4):
    ei = edge_index.astype(jnp.int32)
    src = ei[0]
    dst = ei[1]
    pad = N + (jnp.arange(E_PAD - E, dtype=jnp.int32) % (N_ACC - N))
    src = jnp.concatenate([src, pad])
    dst = jnp.concatenate([dst, pad])
    dst2 = dst.reshape(E_PAD // CH, CH)

    eye8 = jnp.eye(8, dtype=F32)
    w1a = jnp.kron(eye8, W1[:, :16])
    w1b = jnp.kron(eye8, W1[:, 16:])
    w2a = jnp.kron(eye8, W2[:16, :])
    w2b = jnp.kron(eye8, W2[16:, :])
    w3a = jnp.kron(eye8, W3[:, :16])
    w3b = jnp.kron(eye8, W3[:, 16:])
    w4a = jnp.kron(eye8, W4[:16, :])
    w4b = jnp.kron(eye8, W4[16:, :])
    b1a, b1b = jnp.tile(b1[:16], 8), jnp.tile(b1[16:], 8)
    b2t = jnp.tile(b2, 8)
    b3a, b3b = jnp.tile(b3[:16], 8), jnp.tile(b3[16:], 8)
    b4t = jnp.tile(b4, 8)
    x8 = jnp.pad(x.reshape(N * 16 // 128, 128),
                 ((0, NR8 - N * 16 // 128), (0, 0)))

    deg = _sc_deg(dst2).reshape(NC, NR8, 128)

    s8, g1s = _tc_call(
        _tck0_body,
        [_rs2(), _rs(), _fs(), _fs()],
        [_rs(), _rs2()],
        [_p2(), _p2((NC, NR8, 128))],
    )(deg, x8, w1a, w1b)

    y1 = _sc_spmm_wide(src, dst2, g1s.reshape(NC * N_ACC, 16))

    xl1, g2 = _tc_call(
        _tck1_body,
        [_rs2(), _rs2(), _rs(), _bs(), _bs(), _fs(), _fs()],
        [_rs2(), _rs()],
        [_p2((NC, NR8, 128)), _p2()],
    )(y1.reshape(NC, NR8, 128), g1s, s8, b1a, b1b, w2a, w2b)

    y2 = _sc_spmm_narrow(src, dst2, g2.reshape(N_ACC, 16))

    (g3s,) = _tc_call(
        _tck2_body,
        [_rs2(), _rs(), _rs(), _bs(), _fs(), _fs()],
        [_rs2()],
        [_p2((NC, NR8, 128))],
    )(y2.reshape(NC, NR8, 128), g2, s8, b2t, w3a, w3b)

    y3 = _sc_spmm_wide(src, dst2, g3s.reshape(NC * N_ACC, 16))

    (g4,) = _tc_call(
        _tck3_body,
        [_rs2(), _rs2(), _rs(), _bs(), _bs(), _rs2(), _fs(), _fs()],
        [_rs()],
        [_p2()],
    )(y3.reshape(NC, NR8, 128), g3s, s8, b3a, b3b, xl1, w4a, w4b)

    y4 = _sc_spmm_narrow(src, dst2, g4.reshape(N_ACC, 16))

    (out8,) = _tc_call(
        _tck4_body,
        [_rs2(), _rs(), _rs(), _bs(), _rs()],
        [_rs()],
        [_p2()],
    )(y4.reshape(NC, NR8, 128), g4, s8, b4t, x8)

    return out8.reshape(N_ACC, 16)[:N]


# speedup vs baseline: 85.4503x; 1.0036x over previous
import functools

import jax
import jax.numpy as jnp
from jax import lax
from jax.experimental import pallas as pl
from jax.experimental.pallas import tpu as pltpu
from jax.experimental.pallas import tpu_sc as plsc

N = 100000
E = 3200000
E_PAD = 3276800
NC = 2
NS = 16
CH = 128
N_ACC = 102400
RPS = N_ACC // NS
ZR = 160

F32 = jnp.float32


def _sc_mesh():
    return plsc.VectorSubcoreMesh(core_axis_name="c", subcore_axis_name="s",
                                  num_cores=NC, num_subcores=NS)


BLK = 25
BE = BLK * CH
NSLOT = 5


def _zero_acc(zb, acc, s, sem):
    def zrow(i, _):
        zb[i, :] = jnp.zeros((16,), F32)
        return 0

    lax.fori_loop(0, ZR, zrow, 0)

    def zcopy(j, _):
        pltpu.async_copy(zb, acc.at[pl.ds(s * RPS + j * ZR, ZR)], sem)
        return 0

    lax.fori_loop(0, RPS // ZR, zcopy, 0)

    def zwait(j, _):
        pltpu.make_async_copy(zb, acc.at[pl.ds(s * RPS + j * ZR, ZR)],
                              sem).wait()
        return 0

    lax.fori_loop(0, RPS // ZR, zwait, 0)


def _spmm_body(mode, *refs):
    if mode == "deg":
        (dst2_hbm, out_hbm, dstb, rows_v, zb, acc, si0, *sems) = refs
        src_hbm = table_hbm = srcb = None
    else:
        (src_hbm, dst2_hbm, table_hbm, out_hbm,
         srcb, dstb, rows_v, zb, acc, si0, *sems) = refs
    sg = sems[0:NSLOT]
    ss = sems[NSLOT:2 * NSLOT]

    c = lax.axis_index("c")
    s = lax.axis_index("s")
    _zero_acc(zb, acc, s, sems[0])
    if mode == "deg":
        def orow(i, _):
            for p in range(NSLOT):
                rows_v[p, i, :] = jnp.full((16,), 1.0, F32)
            return 0
        lax.fori_loop(0, CH, orow, 0)
    plsc.subcore_barrier()

    if mode == "wide":
        ew = E_PAD // NS
        wbase = s * ew
    else:
        ew = E_PAD // (NC * NS)
        wbase = (c * NS + s) * ew
    nb = ew // BE
    wrow = wbase // CH

    def fire_idx(t, pb):
        if mode != "deg":
            pltpu.async_copy(src_hbm.at[pl.ds(wbase + t * BE, BE)],
                             srcb.at[pb], si0)
        pltpu.async_copy(dst2_hbm.at[pl.ds(wrow + t * BLK, BLK)],
                         dstb.at[pb], si0)

    def wait_idx(t, pb):
        if mode != "deg":
            pltpu.make_async_copy(src_hbm.at[pl.ds(wbase + t * BE, BE)],
                                  srcb.at[pb], si0).wait()
        pltpu.make_async_copy(dst2_hbm.at[pl.ds(wrow + t * BLK, BLK)],
                              dstb.at[pb], si0).wait()

    def fire_scatter(jj, p):
        t = jj // BLK
        pltpu.async_copy(rows_v.at[p], acc.at[dstb.at[t % 2, jj % BLK]],
                         ss[p], add=True)

    def wait_scatter(p):
        pltpu.make_async_copy(rows_v.at[p], acc.at[dstb.at[0, 0]],
                              ss[p]).wait()

    fire_idx(0, 0)

    LAG = 4

    def body(g5, _):
        for p in range(NSLOT):
            jj = g5 * NSLOT + p
            t = jj // BLK
            pb = t % 2
            if p == 0:
                @pl.when(g5 % 5 == 0)
                def _():
                    wait_idx(t, pb)

                @pl.when((g5 % 5 == 1) & (t + 1 < nb))
                def _():
                    fire_idx(t + 1, (t + 1) % 2)

            @pl.when(g5 >= 1)
            def _():
                wait_scatter(p)

            if mode == "wide":
                shift = c * N_ACC
                cbase = (jj % BLK) * CH
                for k in range(CH // 16):
                    sl = pl.ds(cbase + k * 16, 16)
                    srcb[pb, sl] = srcb[pb, sl] + shift

            pltpu.async_copy(
                table_hbm.at[srcb.at[pb, pl.ds((jj % BLK) * CH, CH)]],
                rows_v.at[p], sg[p])

            sp = (p - LAG) % NSLOT
            cond = True if p >= LAG else (g5 >= 1)

            @pl.when(cond)
            def _():
                pltpu.make_async_copy(
                    table_hbm.at[srcb.at[0, pl.ds(0, CH)]],
                    rows_v.at[sp], sg[sp]).wait()
                fire_scatter(jj - LAG, sp)
        return 0

    def body_deg(g5, _):
        for p in range(NSLOT):
            jj = g5 * NSLOT + p
            t = jj // BLK
            if p == 0:
                @pl.when(g5 % 5 == 0)
                def _():
                    wait_idx(t, t % 2)

                @pl.when((g5 % 5 == 1) & (t + 1 < nb))
                def _():
                    fire_idx(t + 1, (t + 1) % 2)

            @pl.when(g5 >= 1)
            def _():
                wait_scatter(p)

            fire_scatter(jj, p)
        return 0

    if mode == "deg":
        lax.fori_loop(0, ew // CH // NSLOT, body_deg, 0)
    else:
        lax.fori_loop(0, ew // CH // NSLOT, body, 0)
        nch = ew // CH
        for k in range(nch - 4, nch):
            pltpu.make_async_copy(table_hbm.at[srcb.at[0, pl.ds(0, CH)]],
                                  rows_v.at[k % NSLOT], sg[k % NSLOT]).wait()
            fire_scatter(k, k % NSLOT)
    for p in range(NSLOT):
        wait_scatter(p)

    plsc.subcore_barrier()
    pltpu.sync_copy(acc.at[pl.ds(s * RPS, RPS)],
                    out_hbm.at[c, pl.ds(s * RPS, RPS)])


def _make_spmm(mode):
    scratch = []
    if mode != "deg":
        scratch.append(pltpu.VMEM((2, BE), jnp.int32))
    scratch += [
        pltpu.VMEM((2, BLK, CH), jnp.int32),
        pltpu.VMEM((NSLOT, CH, 16), F32),
        pltpu.VMEM((ZR, 16), F32),
        pltpu.VMEM_SHARED((N_ACC, 16), F32),
    ]
    scratch += [pltpu.SemaphoreType.DMA] * (1 + 2 * NSLOT)
    return pl.kernel(
        functools.partial(_spmm_body, mode),
        out_type=jax.ShapeDtypeStruct((NC, N_ACC, 16), F32),
        mesh=_sc_mesh(),
        scratch_types=scratch,
        compiler_params=pltpu.CompilerParams(use_tc_tiling_on_sc=False),
    )


_sc_spmm_wide = _make_spmm("wide")
_sc_spmm_narrow = _make_spmm("narrow")
_sc_deg = _make_spmm("deg")


R8 = 3200
NR8 = N_ACC * 16 // 128
GRID8 = NR8 // R8


def _rs():
    return pl.BlockSpec((R8, 128), lambda i: (i, 0))


def _rs2():
    return pl.BlockSpec((NC, R8, 128), lambda i: (0, i, 0))


def _fs():
    return pl.BlockSpec((128, 128), lambda i: (0, 0))


def _bs():
    return pl.BlockSpec((128,), lambda i: (0,))


def _tc_call(body, in_specs, out_specs, out_shapes):
    return pl.pallas_call(
        body,
        grid=(GRID8,),
        in_specs=in_specs,
        out_specs=out_specs,
        out_shape=out_shapes,
    )


def _dot(a, b):
    return jnp.dot(a, b, preferred_element_type=F32)


def _tck0_body(deg_ref, x_ref, wa_ref, wb_ref, s_ref, g1_ref):
    d = deg_ref[0] + deg_ref[1] + 1.0
    sb = lax.rsqrt(d)
    s_ref[...] = sb
    x8 = x_ref[...]
    g1_ref[0] = sb * _dot(x8, wa_ref[...])
    g1_ref[1] = sb * _dot(x8, wb_ref[...])


def _tck1_body(y_ref, g_ref, s_ref, ba_ref, bb_ref, wa_ref, wb_ref,
               xl1_ref, g2_ref):
    sb = s_ref[...]
    xa = jnp.maximum(sb * (y_ref[0] + g_ref[0]) + ba_ref[...][None, :], 0.0)
    xb = jnp.maximum(sb * (y_ref[1] + g_ref[1]) + bb_ref[...][None, :], 0.0)
    xl1_ref[0] = xa
    xl1_ref[1] = xb
    g2_ref[...] = sb * (_dot(xa, wa_ref[...]) + _dot(xb, wb_ref[...]))


def _tck2_body(y_ref, g_ref, s_ref, b_ref, wa_ref, wb_ref, g3_ref):
    sb = s_ref[...]
    x2 = jnp.maximum(sb * (y_ref[0] + y_ref[1] + g_ref[...])
                     + b_ref[...][None, :], 0.0)
    g3_ref[0] = sb * _dot(x2, wa_ref[...])
    g3_ref[1] = sb * _dot(x2, wb_ref[...])


def _tck3_body(y_ref, g_ref, s_ref, ba_ref, bb_ref, xl1_ref, wa_ref, wb_ref,
               g4_ref):
    sb = s_ref[...]
    ta = jnp.maximum(sb * (y_ref[0] + g_ref[0]) + ba_ref[...][None, :],
                     0.0) + xl1_ref[0]
    tb = jnp.maximum(sb * (y_ref[1] + g_ref[1]) + bb_ref[...][None, :],
                     0.0) + xl1_ref[1]
    g4_ref[...] = sb * (_dot(ta, wa_ref[...]) + _dot(tb, wb_ref[...]))


def _tck4_body(y_ref, g_ref, s_ref, b_ref, x_ref, out_ref):
    sb = s_ref[...]
    z = sb * (y_ref[0] + y_ref[1] + g_ref[...]) + b_ref[...][None, :]
    out_ref[...] = jnp.maximum(z, 0.0) + x_ref[...]


def _p2(shape=(NR8, 128)):
    return jax.ShapeDtypeStruct(shape, F32)


def kernel(x, edge_index, W1, b1, W2, b2, W3, b3, W4, b4):
    ei = edge_index.astype(jnp.int32)
    src = ei[0]
    dst = ei[1]
    pad = N + (jnp.arange(E_PAD - E, dtype=jnp.int32) % (N_ACC - N))
    src = jnp.concatenate([src, pad])
    dst = jnp.concatenate([dst, pad])
    dst2 = dst.reshape(E_PAD // CH, CH)

    eye8 = jnp.eye(8, dtype=F32)
    w1a = jnp.kron(eye8, W1[:, :16])
    w1b = jnp.kron(eye8, W1[:, 16:])
    w2a = jnp.kron(eye8, W2[:16, :])
    w2b = jnp.kron(eye8, W2[16:, :])
    w3a = jnp.kron(eye8, W3[:, :16])
    w3b = jnp.kron(eye8, W3[:, 16:])
    w4a = jnp.kron(eye8, W4[:16, :])
    w4b = jnp.kron(eye8, W4[16:, :])
    b1a, b1b = jnp.tile(b1[:16], 8), jnp.tile(b1[16:], 8)
    b2t = jnp.tile(b2, 8)
    b3a, b3b = jnp.tile(b3[:16], 8), jnp.tile(b3[16:], 8)
    b4t = jnp.tile(b4, 8)
    x8 = jnp.pad(x.reshape(N * 16 // 128, 128),
                 ((0, NR8 - N * 16 // 128), (0, 0)))

    deg = _sc_deg(dst2).reshape(NC, NR8, 128)

    s8, g1s = _tc_call(
        _tck0_body,
        [_rs2(), _rs(), _fs(), _fs()],
        [_rs(), _rs2()],
        [_p2(), _p2((NC, NR8, 128))],
    )(deg, x8, w1a, w1b)

    y1 = _sc_spmm_wide(src, dst2, g1s.reshape(NC * N_ACC, 16))

    xl1, g2 = _tc_call(
        _tck1_body,
        [_rs2(), _rs2(), _rs(), _bs(), _bs(), _fs(), _fs()],
        [_rs2(), _rs()],
        [_p2((NC, NR8, 128)), _p2()],
    )(y1.reshape(NC, NR8, 128), g1s, s8, b1a, b1b, w2a, w2b)

    y2 = _sc_spmm_narrow(src, dst2, g2.reshape(N_ACC, 16))

    (g3s,) = _tc_call(
        _tck2_body,
        [_rs2(), _rs(), _rs(), _bs(), _fs(), _fs()],
        [_rs2()],
        [_p2((NC, NR8, 128))],
    )(y2.reshape(NC, NR8, 128), g2, s8, b2t, w3a, w3b)

    y3 = _sc_spmm_wide(src, dst2, g3s.reshape(NC * N_ACC, 16))

    (g4,) = _tc_call(
        _tck3_body,
        [_rs2(), _rs2(), _rs(), _bs(), _bs(), _rs2(), _fs(), _fs()],
        [_rs()],
        [_p2()],
    )(y3.reshape(NC, NR8, 128), g3s, s8, b3a, b3b, xl1, w4a, w4b)

    y4 = _sc_spmm_narrow(src, dst2, g4.reshape(N_ACC, 16))

    (out8,) = _tc_call(
        _tck4_body,
        [_rs2(), _rs(), _rs(), _bs(), _rs()],
        [_rs()],
        [_p2()],
    )(y4.reshape(NC, NR8, 128), g4, s8, b4t, x8)

    return out8.reshape(N_ACC, 16)[:N]
